# baseline (speedup 1.0000x reference)
import functools

import jax
import jax.numpy as jnp
from jax import lax
from jax.experimental import pallas as pl
from jax.experimental.pallas import tpu as pltpu

BN_EPS = 1e-5


def _conv_stats_kernel(x_ref, w_ref, y_ref, sum_ref, sq_ref, col_ref, *,
                       kh, kw, w_in, wo, ho, p_img, slab):
    x2d = x_ref[0].reshape(x_ref.shape[1], slab).astype(jnp.bfloat16)
    cin = x2d.shape[0]
    cout = y_ref.shape[1]
    for i in range(kh):
        for j in range(kw):
            t = i * kw + j
            off = i * w_in + j
            if off + p_img <= slab:
                sl = x2d[:, off:off + p_img]
            else:
                extra = off + p_img - slab
                sl = jnp.concatenate([x2d[:, off:], x2d[:, :extra]], axis=1)
            col_ref[t * cin:(t + 1) * cin, :] = sl
    y = jnp.dot(w_ref[...], col_ref[...],
                preferred_element_type=jnp.float32)
    yb = y.astype(jnp.bfloat16)
    y_ref[0] = yb
    lane = lax.broadcasted_iota(jnp.int32, (1, p_img), 1)
    vmask = ((lane % w_in) < wo).astype(jnp.bfloat16)
    ym = yb * vmask
    aug = jnp.concatenate(
        [ym, jnp.ones((1, p_img), jnp.bfloat16)], axis=0)
    g = lax.dot_general(aug, aug, (((1,), (1,)), ((), ())),
                        preferred_element_type=jnp.float32)
    sum_ref[0] = g[:cout, cout:cout + 1]
    row = lax.broadcasted_iota(jnp.int32, (cout, cout + 1), 0)
    coli = lax.broadcasted_iota(jnp.int32, (cout, cout + 1), 1)
    sq_ref[0] = jnp.sum(jnp.where(row == coli, g[:cout, :], 0.0),
                        axis=1, keepdims=True)


def _bn_relu_kernel(y_ref, sum_ref, sq_ref, gb_ref, out_ref, *,
                    inv_p, w_in, wo, ho):
    s1 = jnp.sum(sum_ref[...], axis=0)
    s2 = jnp.sum(sq_ref[...], axis=0)
    mean = s1 * inv_p
    var = jnp.maximum(s2 * inv_p - mean * mean, 0.0)
    scale = gb_ref[:, 0:1] * lax.rsqrt(var + BN_EPS)
    shift = gb_ref[:, 1:2] - mean * scale
    nb = y_ref.shape[0]
    for b in range(nb):
        for r in range(ho):
            zr = y_ref[b, :, r * w_in:r * w_in + wo].astype(jnp.float32)
            out_ref[b, :, r * wo:(r + 1) * wo] = jnp.maximum(
                zr * scale + shift, 0.0)


@jax.jit
def kernel(x, conv_w, conv_b, bn_gamma, bn_beta):
    del conv_b
    n, cin, h, w_in = x.shape
    cout, cin2, kh, kw = conv_w.shape
    assert cin2 == cin
    ho = h - kh + 1
    wo = w_in - kw + 1
    slab = h * w_in
    p_img = ho * w_in
    p_out = ho * wo
    k_dim = kh * kw * cin

    w_mat = conv_w.transpose(0, 2, 3, 1).reshape(cout, k_dim)
    w_mat = w_mat.astype(jnp.bfloat16)
    gb = jnp.stack([bn_gamma, bn_beta], axis=1)

    cparams = pltpu.CompilerParams(
        dimension_semantics=("parallel",),
        vmem_limit_bytes=64 * 1024 * 1024)

    y, psum, psq = pl.pallas_call(
        functools.partial(_conv_stats_kernel, kh=kh, kw=kw, w_in=w_in,
                          wo=wo, ho=ho, p_img=p_img, slab=slab),
        out_shape=(
            jax.ShapeDtypeStruct((n, cout, p_img), jnp.bfloat16),
            jax.ShapeDtypeStruct((n, cout, 1), jnp.float32),
            jax.ShapeDtypeStruct((n, cout, 1), jnp.float32),
        ),
        grid=(n,),
        in_specs=[
            pl.BlockSpec((1, cin, h, w_in), lambda i: (i, 0, 0, 0)),
            pl.BlockSpec((cout, k_dim), lambda i: (0, 0)),
        ],
        out_specs=(
            pl.BlockSpec((1, cout, p_img), lambda i: (i, 0, 0)),
            pl.BlockSpec((1, cout, 1), lambda i: (i, 0, 0)),
            pl.BlockSpec((1, cout, 1), lambda i: (i, 0, 0)),
        ),
        scratch_shapes=[pltpu.VMEM((k_dim, p_img), jnp.bfloat16)],
        compiler_params=cparams,
    )(x, w_mat)

    nb = 4
    while n % nb:
        nb -= 1
    out_p = pl.pallas_call(
        functools.partial(_bn_relu_kernel, inv_p=1.0 / (n * ho * wo),
                          w_in=w_in, wo=wo, ho=ho),
        out_shape=jax.ShapeDtypeStruct((n, cout, p_out), jnp.float32),
        grid=(n // nb,),
        in_specs=[
            pl.BlockSpec((nb, cout, p_img), lambda i: (i, 0, 0)),
            pl.BlockSpec((n, cout, 1), lambda i: (0, 0, 0)),
            pl.BlockSpec((n, cout, 1), lambda i: (0, 0, 0)),
            pl.BlockSpec((cout, 2), lambda i: (0, 0)),
        ],
        out_specs=pl.BlockSpec((nb, cout, p_out), lambda i: (i, 0, 0)),
        compiler_params=cparams,
    )(y, psum, psq, gb)

    return out_p.reshape(n, cout, ho, wo)


# speedup vs baseline: 1.2779x; 1.2779x over previous
import functools

import jax
import jax.numpy as jnp
from jax import lax
from jax.experimental import pallas as pl
from jax.experimental.pallas import tpu as pltpu

BN_EPS = 1e-5


def _conv_stats_kernel(x_ref, w_ref, y_ref, sum_ref, sq_ref, col_ref, *,
                       kh, kw, w_in, wo, ho, p_img, slab):
    nb = x_ref.shape[0]
    cin = x_ref.shape[1]
    cout = y_ref.shape[1]
    lane = lax.broadcasted_iota(jnp.int32, (1, p_img), 1)
    vmask = ((lane % w_in) < wo).astype(jnp.bfloat16)
    for b in range(nb):
        x2d = x_ref[b].astype(jnp.bfloat16)
        for i in range(kh):
            for j in range(kw):
                t = i * kw + j
                off = i * w_in + j
                if off + p_img <= slab:
                    sl = x2d[:, off:off + p_img]
                else:
                    extra = off + p_img - slab
                    sl = jnp.concatenate([x2d[:, off:], x2d[:, :extra]],
                                         axis=1)
                col_ref[b % 2, t * cin:(t + 1) * cin, :] = sl
        y = jnp.dot(w_ref[...], col_ref[b % 2],
                    preferred_element_type=jnp.float32)
        yb = y.astype(jnp.bfloat16)
        y_ref[b] = yb
        ym = yb * vmask
        aug = jnp.concatenate(
            [ym, jnp.ones((1, p_img), jnp.bfloat16)], axis=0)
        g = lax.dot_general(aug, aug, (((1,), (1,)), ((), ())),
                            preferred_element_type=jnp.float32)
        sum_ref[b] = g[:cout, cout:cout + 1]
        row = lax.broadcasted_iota(jnp.int32, (cout, cout + 1), 0)
        coli = lax.broadcasted_iota(jnp.int32, (cout, cout + 1), 1)
        sq_ref[b] = jnp.sum(jnp.where(row == coli, g[:cout, :], 0.0),
                            axis=1, keepdims=True)


def _bn_relu_kernel(y_ref, sum_ref, sq_ref, gb_ref, out_ref, *,
                    inv_p, w_in, wo, ho):
    s1 = jnp.sum(sum_ref[...], axis=0)
    s2 = jnp.sum(sq_ref[...], axis=0)
    mean = s1 * inv_p
    var = jnp.maximum(s2 * inv_p - mean * mean, 0.0)
    scale = gb_ref[:, 0:1] * lax.rsqrt(var + BN_EPS)
    shift = gb_ref[:, 1:2] - mean * scale
    nb = y_ref.shape[0]
    for b in range(nb):
        for r in range(ho):
            zr = y_ref[b, :, r * w_in:r * w_in + wo].astype(jnp.float32)
            out_ref[b, :, r * wo:(r + 1) * wo] = jnp.maximum(
                zr * scale + shift, 0.0)


@jax.jit
def kernel(x, conv_w, conv_b, bn_gamma, bn_beta):
    del conv_b
    n, cin, h, w_in = x.shape
    cout, cin2, kh, kw = conv_w.shape
    assert cin2 == cin
    ho = h - kh + 1
    wo = w_in - kw + 1
    slab = h * w_in
    p_img = ho * w_in
    p_out = ho * wo
    k_dim = kh * kw * cin

    x3 = x.reshape(n, cin, slab)
    w_mat = conv_w.transpose(0, 2, 3, 1).reshape(cout, k_dim)
    w_mat = w_mat.astype(jnp.bfloat16)
    gb = jnp.stack([bn_gamma, bn_beta], axis=1)

    cparams = pltpu.CompilerParams(
        dimension_semantics=("parallel",),
        vmem_limit_bytes=64 * 1024 * 1024)

    nb1 = 2
    while n % nb1:
        nb1 -= 1
    y, psum, psq = pl.pallas_call(
        functools.partial(_conv_stats_kernel, kh=kh, kw=kw, w_in=w_in,
                          wo=wo, ho=ho, p_img=p_img, slab=slab),
        out_shape=(
            jax.ShapeDtypeStruct((n, cout, p_img), jnp.bfloat16),
            jax.ShapeDtypeStruct((n, cout, 1), jnp.float32),
            jax.ShapeDtypeStruct((n, cout, 1), jnp.float32),
        ),
        grid=(n // nb1,),
        in_specs=[
            pl.BlockSpec((nb1, cin, slab), lambda i: (i, 0, 0)),
            pl.BlockSpec((cout, k_dim), lambda i: (0, 0)),
        ],
        out_specs=(
            pl.BlockSpec((nb1, cout, p_img), lambda i: (i, 0, 0)),
            pl.BlockSpec((nb1, cout, 1), lambda i: (i, 0, 0)),
            pl.BlockSpec((nb1, cout, 1), lambda i: (i, 0, 0)),
        ),
        scratch_shapes=[pltpu.VMEM((2, k_dim, p_img), jnp.bfloat16)],
        compiler_params=cparams,
    )(x3, w_mat)

    nb = 4
    while n % nb:
        nb -= 1
    out_p = pl.pallas_call(
        functools.partial(_bn_relu_kernel, inv_p=1.0 / (n * ho * wo),
                          w_in=w_in, wo=wo, ho=ho),
        out_shape=jax.ShapeDtypeStruct((n, cout, p_out), jnp.float32),
        grid=(n // nb,),
        in_specs=[
            pl.BlockSpec((nb, cout, p_img), lambda i: (i, 0, 0)),
            pl.BlockSpec((n, cout, 1), lambda i: (0, 0, 0)),
            pl.BlockSpec((n, cout, 1), lambda i: (0, 0, 0)),
            pl.BlockSpec((cout, 2), lambda i: (0, 0)),
        ],
        out_specs=pl.BlockSpec((nb, cout, p_out), lambda i: (i, 0, 0)),
        compiler_params=cparams,
    )(y, psum, psq, gb)

    return out_p.reshape(n, cout, ho, wo)


# speedup vs baseline: 1.3343x; 1.0441x over previous
import functools

import jax
import jax.numpy as jnp
from jax import lax
from jax.experimental import pallas as pl
from jax.experimental.pallas import tpu as pltpu

BN_EPS = 1e-5


def _conv_stats_kernel(x_ref, w_ref, y_ref, sum_ref, sq_ref, col_ref, *,
                       kh, kw, w_in, wo, ho, p_img, slab):
    nb = x_ref.shape[0]
    cin = x_ref.shape[1]
    cout = y_ref.shape[1]
    lane = lax.broadcasted_iota(jnp.int32, (1, p_img), 1)
    vmask = ((lane % w_in) < wo).astype(jnp.bfloat16)
    for b in range(nb):
        x2d = x_ref[b].astype(jnp.bfloat16)
        for i in range(kh):
            for j in range(kw):
                t = i * kw + j
                off = i * w_in + j
                if off + p_img <= slab:
                    sl = x2d[:, off:off + p_img]
                else:
                    extra = off + p_img - slab
                    sl = jnp.concatenate([x2d[:, off:], x2d[:, :extra]],
                                         axis=1)
                col_ref[b % 2, t * cin:(t + 1) * cin, :] = sl
        y = jnp.dot(w_ref[...], col_ref[b % 2],
                    preferred_element_type=jnp.float32)
        yb = y.astype(jnp.bfloat16)
        y_ref[b] = yb
        ym = yb * vmask
        aug = jnp.concatenate(
            [ym, jnp.ones((1, p_img), jnp.bfloat16)], axis=0)
        g = lax.dot_general(aug, aug, (((1,), (1,)), ((), ())),
                            preferred_element_type=jnp.float32)
        sum_ref[b] = g[:cout, cout:cout + 1]
        row = lax.broadcasted_iota(jnp.int32, (cout, cout + 1), 0)
        coli = lax.broadcasted_iota(jnp.int32, (cout, cout + 1), 1)
        sq_ref[b] = jnp.sum(jnp.where(row == coli, g[:cout, :], 0.0),
                            axis=1, keepdims=True)


def _bn_relu_kernel(y_ref, sum_ref, sq_ref, gb_ref, out_ref, *,
                    inv_p, w_in, wo, ho):
    s1 = jnp.sum(sum_ref[...], axis=0)
    s2 = jnp.sum(sq_ref[...], axis=0)
    mean = s1 * inv_p
    var = jnp.maximum(s2 * inv_p - mean * mean, 0.0)
    scale = gb_ref[:, 0:1] * lax.rsqrt(var + BN_EPS)
    shift = gb_ref[:, 1:2] - mean * scale
    nb = y_ref.shape[0]
    for b in range(nb):
        for r in range(ho):
            zr = y_ref[b, :, r * w_in:r * w_in + wo].astype(jnp.float32)
            out_ref[b, :, r * wo:(r + 1) * wo] = jnp.maximum(
                zr * scale + shift, 0.0)


@jax.jit
def kernel(x, conv_w, conv_b, bn_gamma, bn_beta):
    del conv_b
    n, cin, h, w_in = x.shape
    cout, cin2, kh, kw = conv_w.shape
    assert cin2 == cin
    ho = h - kh + 1
    wo = w_in - kw + 1
    slab = h * w_in
    p_img = ho * w_in
    p_out = ho * wo
    k_dim = kh * kw * cin

    x3 = x.reshape(n, cin, slab)
    w_mat = conv_w.transpose(0, 2, 3, 1).reshape(cout, k_dim)
    w_mat = w_mat.astype(jnp.bfloat16)
    gb = jnp.stack([bn_gamma, bn_beta], axis=1)

    cparams = pltpu.CompilerParams(
        dimension_semantics=("parallel",),
        vmem_limit_bytes=64 * 1024 * 1024)

    nb1 = 4
    while n % nb1:
        nb1 -= 1
    y, psum, psq = pl.pallas_call(
        functools.partial(_conv_stats_kernel, kh=kh, kw=kw, w_in=w_in,
                          wo=wo, ho=ho, p_img=p_img, slab=slab),
        out_shape=(
            jax.ShapeDtypeStruct((n, cout, p_img), jnp.bfloat16),
            jax.ShapeDtypeStruct((n, cout, 1), jnp.float32),
            jax.ShapeDtypeStruct((n, cout, 1), jnp.float32),
        ),
        grid=(n // nb1,),
        in_specs=[
            pl.BlockSpec((nb1, cin, slab), lambda i: (i, 0, 0)),
            pl.BlockSpec((cout, k_dim), lambda i: (0, 0)),
        ],
        out_specs=(
            pl.BlockSpec((nb1, cout, p_img), lambda i: (i, 0, 0)),
            pl.BlockSpec((nb1, cout, 1), lambda i: (i, 0, 0)),
            pl.BlockSpec((nb1, cout, 1), lambda i: (i, 0, 0)),
        ),
        scratch_shapes=[pltpu.VMEM((2, k_dim, p_img), jnp.bfloat16)],
        compiler_params=cparams,
    )(x3, w_mat)

    nb = 8
    while n % nb:
        nb -= 1
    out_p = pl.pallas_call(
        functools.partial(_bn_relu_kernel, inv_p=1.0 / (n * ho * wo),
                          w_in=w_in, wo=wo, ho=ho),
        out_shape=jax.ShapeDtypeStruct((n, cout, p_out), jnp.float32),
        grid=(n // nb,),
        in_specs=[
            pl.BlockSpec((nb, cout, p_img), lambda i: (i, 0, 0)),
            pl.BlockSpec((n, cout, 1), lambda i: (0, 0, 0)),
            pl.BlockSpec((n, cout, 1), lambda i: (0, 0, 0)),
            pl.BlockSpec((cout, 2), lambda i: (0, 0)),
        ],
        out_specs=pl.BlockSpec((nb, cout, p_out), lambda i: (i, 0, 0)),
        compiler_params=cparams,
    )(y, psum, psq, gb)

    return out_p.reshape(n, cout, ho, wo)


# speedup vs baseline: 1.3458x; 1.0086x over previous
import functools

import jax
import jax.numpy as jnp
from jax import lax
from jax.experimental import pallas as pl
from jax.experimental.pallas import tpu as pltpu

BN_EPS = 1e-5


def _conv_stats_kernel(x_ref, w_ref, y_ref, sum_ref, sq_ref, col_ref, *,
                       kh, kw, w_in, wo, ho, p_img, slab):
    nb = x_ref.shape[0]
    cin = x_ref.shape[1]
    cout = y_ref.shape[1]
    lane = lax.broadcasted_iota(jnp.int32, (1, p_img), 1)
    vmask = ((lane % w_in) < wo).astype(jnp.bfloat16)
    for b in range(nb):
        x2d = x_ref[b].astype(jnp.bfloat16)
        for i in range(kh):
            for j in range(kw):
                t = i * kw + j
                off = i * w_in + j
                if off + p_img <= slab:
                    sl = x2d[:, off:off + p_img]
                else:
                    extra = off + p_img - slab
                    sl = jnp.concatenate([x2d[:, off:], x2d[:, :extra]],
                                         axis=1)
                col_ref[b % 2, t * cin:(t + 1) * cin, :] = sl
        y = jnp.dot(w_ref[...], col_ref[b % 2],
                    preferred_element_type=jnp.float32)
        yb = y.astype(jnp.bfloat16)
        y_ref[b] = yb
        ym = yb * vmask
        aug = jnp.concatenate(
            [ym, jnp.ones((1, p_img), jnp.bfloat16)], axis=0)
        g = lax.dot_general(aug, aug, (((1,), (1,)), ((), ())),
                            preferred_element_type=jnp.float32)
        sum_ref[b] = g[:cout, cout:cout + 1]
        row = lax.broadcasted_iota(jnp.int32, (cout, cout + 1), 0)
        coli = lax.broadcasted_iota(jnp.int32, (cout, cout + 1), 1)
        sq_ref[b] = jnp.sum(jnp.where(row == coli, g[:cout, :], 0.0),
                            axis=1, keepdims=True)


def _bn_relu_kernel(y_ref, sum_ref, sq_ref, gb_ref, out_ref, *,
                    inv_p, w_in, wo, ho):
    s1 = jnp.sum(sum_ref[...], axis=0)
    s2 = jnp.sum(sq_ref[...], axis=0)
    mean = s1 * inv_p
    var = jnp.maximum(s2 * inv_p - mean * mean, 0.0)
    scale = gb_ref[:, 0:1] * lax.rsqrt(var + BN_EPS)
    shift = gb_ref[:, 1:2] - mean * scale
    nb = y_ref.shape[0]
    for b in range(nb):
        for r in range(ho):
            zr = y_ref[b, :, r * w_in:r * w_in + wo].astype(jnp.float32)
            out_ref[b, :, r * wo:(r + 1) * wo] = jnp.maximum(
                zr * scale + shift, 0.0)


@jax.jit
def kernel(x, conv_w, conv_b, bn_gamma, bn_beta):
    del conv_b
    n, cin, h, w_in = x.shape
    cout, cin2, kh, kw = conv_w.shape
    assert cin2 == cin
    ho = h - kh + 1
    wo = w_in - kw + 1
    slab = h * w_in
    p_img = ho * w_in
    p_out = ho * wo
    k_dim = kh * kw * cin

    x3 = x.reshape(n, cin, slab)
    w_mat = conv_w.transpose(0, 2, 3, 1).reshape(cout, k_dim)
    w_mat = w_mat.astype(jnp.bfloat16)
    gb = jnp.stack([bn_gamma, bn_beta], axis=1)

    cparams = pltpu.CompilerParams(
        dimension_semantics=("parallel",),
        vmem_limit_bytes=64 * 1024 * 1024)

    nb1 = 8
    while n % nb1:
        nb1 -= 1
    y, psum, psq = pl.pallas_call(
        functools.partial(_conv_stats_kernel, kh=kh, kw=kw, w_in=w_in,
                          wo=wo, ho=ho, p_img=p_img, slab=slab),
        out_shape=(
            jax.ShapeDtypeStruct((n, cout, p_img), jnp.bfloat16),
            jax.ShapeDtypeStruct((n, cout, 1), jnp.float32),
            jax.ShapeDtypeStruct((n, cout, 1), jnp.float32),
        ),
        grid=(n // nb1,),
        in_specs=[
            pl.BlockSpec((nb1, cin, slab), lambda i: (i, 0, 0)),
            pl.BlockSpec((cout, k_dim), lambda i: (0, 0)),
        ],
        out_specs=(
            pl.BlockSpec((nb1, cout, p_img), lambda i: (i, 0, 0)),
            pl.BlockSpec((nb1, cout, 1), lambda i: (i, 0, 0)),
            pl.BlockSpec((nb1, cout, 1), lambda i: (i, 0, 0)),
        ),
        scratch_shapes=[pltpu.VMEM((2, k_dim, p_img), jnp.bfloat16)],
        compiler_params=cparams,
    )(x3, w_mat)

    nb = 8
    while n % nb:
        nb -= 1
    out_p = pl.pallas_call(
        functools.partial(_bn_relu_kernel, inv_p=1.0 / (n * ho * wo),
                          w_in=w_in, wo=wo, ho=ho),
        out_shape=jax.ShapeDtypeStruct((n, cout, p_out), jnp.float32),
        grid=(n // nb,),
        in_specs=[
            pl.BlockSpec((nb, cout, p_img), lambda i: (i, 0, 0)),
            pl.BlockSpec((n, cout, 1), lambda i: (0, 0, 0)),
            pl.BlockSpec((n, cout, 1), lambda i: (0, 0, 0)),
            pl.BlockSpec((cout, 2), lambda i: (0, 0)),
        ],
        out_specs=pl.BlockSpec((nb, cout, p_out), lambda i: (i, 0, 0)),
        compiler_params=cparams,
    )(y, psum, psq, gb)

    return out_p.reshape(n, cout, ho, wo)
